# baseline (speedup 1.0000x reference)
import functools

import jax
import jax.numpy as jnp
from jax.experimental import pallas as pl
from jax.experimental.pallas import tpu as pltpu

_EPS = 1e-08
_K = 32
_MAXI = 2 ** 30


def _knn_kernel(q_ref, k_ref, v_ref, out_ref,
                dm_ref, rd_ref, ri_ref, rv_ref,
                *, n_valid, chunk, n_chunks):
    c = pl.program_id(1)
    qb = q_ref.shape[0]

    @pl.when(c == 0)
    def _init():
        rd_ref[...] = jnp.full_like(rd_ref, jnp.inf)
        ri_ref[...] = -1 - jax.lax.broadcasted_iota(jnp.int32, ri_ref.shape, 1)
        rv_ref[...] = jnp.zeros_like(rv_ref)

    q = q_ref[...]
    kc = k_ref[...]
    qq = jnp.sum(q * q, axis=1)[:, None]
    kk = jnp.sum(kc * kc, axis=1)[None, :]
    dot = jax.lax.dot_general(q, kc, (((1,), (1,)), ((), ())),
                              preferred_element_type=jnp.float32)
    sq = qq + kk - 2.0 * dot
    d = jnp.sqrt(jnp.maximum(sq, 1e-12))
    gidx = c * chunk + jax.lax.broadcasted_iota(jnp.int32, (qb, chunk), 1)
    d = jnp.where(gidx < n_valid, d, jnp.inf)
    dm_ref[...] = d

    vrow = v_ref[0]

    def _insert(mc, gi, vsel, improve):
        rd = rd_ref[...]
        ri = ri_ref[...]
        tau = jnp.max(rd, axis=1)
        imp = improve & (mc < tau)
        sl = jnp.max(jnp.where(rd == tau[:, None], ri, -_MAXI), axis=1)
        ev = (rd == tau[:, None]) & (ri == sl[:, None]) & imp[:, None]
        rd_ref[...] = jnp.where(ev, mc[:, None], rd)
        ri_ref[...] = jnp.where(ev, gi[:, None], ri)
        rv_ref[...] = jnp.where(ev, vsel[:, None], rv_ref[...])

    def cond(go):
        return go == 1

    def body(_):
        dm = dm_ref[...]
        mc = jnp.min(dm, axis=1)
        tau = jnp.max(rd_ref[...], axis=1)
        improve = mc < tau
        anyimp = jnp.sum(improve.astype(jnp.int32)) > 0

        @pl.when(anyimp)
        def _extract():
            gi = jnp.min(jnp.where(dm == mc[:, None], gidx, _MAXI), axis=1)
            hit = gidx == gi[:, None]
            vsel = jnp.sum(jnp.where(hit, vrow, 0.0), axis=1)
            dm2 = jnp.where(hit, jnp.inf, dm)
            mc2 = jnp.min(dm2, axis=1)
            gi2 = jnp.min(jnp.where(dm2 == mc2[:, None], gidx, _MAXI), axis=1)
            hit2 = gidx == gi2[:, None]
            vsel2 = jnp.sum(jnp.where(hit2, vrow, 0.0), axis=1)

            _insert(mc, gi, vsel, improve)
            _insert(mc2, gi2, vsel2, mc2 < jnp.inf)
            dm_ref[...] = jnp.where(hit2, jnp.inf, dm2)

        return anyimp.astype(jnp.int32)

    jax.lax.while_loop(cond, body, jnp.int32(1))

    @pl.when(c == n_chunks - 1)
    def _finish():
        rd = rd_ref[...]
        w = rd / (rd + _EPS)
        out_ref[...] = (jnp.sum(w * rv_ref[...], axis=1) / float(_K))[:, None]


def _impl(queries, keys, values, chunk, qb, interpret=False):
    b, dim = queries.shape
    n = keys.shape[0]
    n_chunks = -(-n // chunk)
    n_pad = n_chunks * chunk
    kp = jnp.pad(keys, ((0, n_pad - n), (0, 0)))
    vp = jnp.pad(values[:, 0], (0, n_pad - n)).reshape(n_chunks, 1, chunk)
    n_qb = b // qb

    kfn = functools.partial(_knn_kernel, n_valid=n, chunk=chunk,
                            n_chunks=n_chunks)
    out = pl.pallas_call(
        kfn,
        grid=(n_qb, n_chunks),
        in_specs=[
            pl.BlockSpec((qb, dim), lambda q, c: (q, 0)),
            pl.BlockSpec((chunk, dim), lambda q, c: (c, 0)),
            pl.BlockSpec((1, 1, chunk), lambda q, c: (c, 0, 0)),
        ],
        out_specs=pl.BlockSpec((qb, 1), lambda q, c: (q, 0)),
        out_shape=jax.ShapeDtypeStruct((b, 1), jnp.float32),
        scratch_shapes=[
            pltpu.VMEM((qb, chunk), jnp.float32),
            pltpu.VMEM((qb, _K), jnp.float32),
            pltpu.VMEM((qb, _K), jnp.int32),
            pltpu.VMEM((qb, _K), jnp.float32),
        ],
        compiler_params=pltpu.CompilerParams(
            dimension_semantics=("parallel", "arbitrary")),
        interpret=interpret,
    )(queries, kp, vp)
    return out


def kernel(queries, keys, values):
    return _impl(queries, keys, values, chunk=2048, qb=1024)


# speedup vs baseline: 1.0003x; 1.0003x over previous
import functools

import jax
import jax.numpy as jnp
from jax.experimental import pallas as pl
from jax.experimental.pallas import tpu as pltpu

_EPS = 1e-08
_K = 32
_MAXI = 2 ** 30


def _knn_kernel(q_ref, k_ref, v_ref, out_ref,
                dm_ref, rd_ref, ri_ref, rv_ref,
                *, n_valid, chunk, n_chunks):
    c = pl.program_id(1)
    qb = q_ref.shape[0]

    @pl.when(c == 0)
    def _init():
        rd_ref[...] = jnp.full_like(rd_ref, jnp.inf)
        ri_ref[...] = -1 - jax.lax.broadcasted_iota(jnp.int32, ri_ref.shape, 1)
        rv_ref[...] = jnp.zeros_like(rv_ref)

    q = q_ref[...]
    kc = k_ref[...]
    qq = jnp.sum(q * q, axis=1)[:, None]
    kk = jnp.sum(kc * kc, axis=1)[None, :]
    dot = jax.lax.dot_general(q, kc, (((1,), (1,)), ((), ())),
                              preferred_element_type=jnp.float32)
    sq = qq + kk - 2.0 * dot
    d = jnp.sqrt(jnp.maximum(sq, 1e-12))
    gidx = c * chunk + jax.lax.broadcasted_iota(jnp.int32, (qb, chunk), 1)
    d = jnp.where(gidx < n_valid, d, jnp.inf)
    dm_ref[...] = d

    vrow = v_ref[0]

    def _insert(mc, gi, vsel, improve):
        rd = rd_ref[...]
        ri = ri_ref[...]
        tau = jnp.max(rd, axis=1)
        imp = improve & (mc < tau)
        sl = jnp.max(jnp.where(rd == tau[:, None], ri, -_MAXI), axis=1)
        ev = (rd == tau[:, None]) & (ri == sl[:, None]) & imp[:, None]
        rd_ref[...] = jnp.where(ev, mc[:, None], rd)
        ri_ref[...] = jnp.where(ev, gi[:, None], ri)
        rv_ref[...] = jnp.where(ev, vsel[:, None], rv_ref[...])

    def cond(go):
        return go == 1

    def body(_):
        dm = dm_ref[...]
        mc = jnp.min(dm, axis=1)
        tau = jnp.max(rd_ref[...], axis=1)
        improve = mc < tau
        anyimp = jnp.sum(improve.astype(jnp.int32)) > 0

        @pl.when(anyimp)
        def _extract():
            gi = jnp.min(jnp.where(dm == mc[:, None], gidx, _MAXI), axis=1)
            hit = gidx == gi[:, None]
            vsel = jnp.sum(jnp.where(hit, vrow, 0.0), axis=1)
            _insert(mc, gi, vsel, improve)
            dm_ref[...] = jnp.where(hit, jnp.inf, dm)

        return anyimp.astype(jnp.int32)

    jax.lax.while_loop(cond, body, jnp.int32(1))

    @pl.when(c == n_chunks - 1)
    def _finish():
        rd = rd_ref[...]
        w = rd / (rd + _EPS)
        out_ref[...] = (jnp.sum(w * rv_ref[...], axis=1) / float(_K))[:, None]


def _impl(queries, keys, values, chunk, qb, interpret=False):
    b, dim = queries.shape
    n = keys.shape[0]
    n_chunks = -(-n // chunk)
    n_pad = n_chunks * chunk
    kp = jnp.pad(keys, ((0, n_pad - n), (0, 0)))
    vp = jnp.pad(values[:, 0], (0, n_pad - n)).reshape(n_chunks, 1, chunk)
    n_qb = b // qb

    kfn = functools.partial(_knn_kernel, n_valid=n, chunk=chunk,
                            n_chunks=n_chunks)
    out = pl.pallas_call(
        kfn,
        grid=(n_qb, n_chunks),
        in_specs=[
            pl.BlockSpec((qb, dim), lambda q, c: (q, 0)),
            pl.BlockSpec((chunk, dim), lambda q, c: (c, 0)),
            pl.BlockSpec((1, 1, chunk), lambda q, c: (c, 0, 0)),
        ],
        out_specs=pl.BlockSpec((qb, 1), lambda q, c: (q, 0)),
        out_shape=jax.ShapeDtypeStruct((b, 1), jnp.float32),
        scratch_shapes=[
            pltpu.VMEM((qb, chunk), jnp.float32),
            pltpu.VMEM((qb, _K), jnp.float32),
            pltpu.VMEM((qb, _K), jnp.int32),
            pltpu.VMEM((qb, _K), jnp.float32),
        ],
        compiler_params=pltpu.CompilerParams(
            dimension_semantics=("parallel", "arbitrary")),
        interpret=interpret,
    )(queries, kp, vp)
    return out


def kernel(queries, keys, values):
    return _impl(queries, keys, values, chunk=2048, qb=1024)


# speedup vs baseline: 1.0422x; 1.0419x over previous
import functools

import jax
import jax.numpy as jnp
from jax.experimental import pallas as pl
from jax.experimental.pallas import tpu as pltpu

_EPS = 1e-08
_K = 32
_MAXI = 2 ** 30


def _knn_kernel(q_ref, k_ref, v_ref, out_ref,
                dm_ref, rd_ref, ri_ref, rv_ref,
                *, n_valid, chunk, n_chunks):
    c = pl.program_id(1)
    qb = q_ref.shape[0]

    @pl.when(c == 0)
    def _init():
        rd_ref[...] = jnp.full_like(rd_ref, jnp.inf)
        ri_ref[...] = -1 - jax.lax.broadcasted_iota(jnp.int32, ri_ref.shape, 1)
        rv_ref[...] = jnp.zeros_like(rv_ref)

    q = q_ref[...]
    kc = k_ref[...]
    qq = jnp.sum(q * q, axis=1)[:, None]
    kk = jnp.sum(kc * kc, axis=1)[None, :]
    dot = jax.lax.dot_general(q, kc, (((1,), (1,)), ((), ())),
                              preferred_element_type=jnp.float32)
    sq = qq + kk - 2.0 * dot
    d = jnp.sqrt(jnp.maximum(sq, 1e-12))
    gidx = c * chunk + jax.lax.broadcasted_iota(jnp.int32, (qb, chunk), 1)
    d = jnp.where(gidx < n_valid, d, jnp.inf)
    dm_ref[...] = d

    vrow = v_ref[0]

    def cond(go):
        return go == 1

    def body(_):
        dm = dm_ref[...]
        mc = jnp.min(dm, axis=1)
        tau = jnp.max(rd_ref[...], axis=1)
        improve = mc < tau
        anyimp = jnp.sum(improve.astype(jnp.int32)) > 0

        @pl.when(anyimp)
        def _extract():
            gi = jnp.min(jnp.where(dm == mc[:, None], gidx, _MAXI), axis=1)
            hit = gidx == gi[:, None]
            vsel = jnp.sum(jnp.where(hit, vrow, 0.0), axis=1)

            rd = rd_ref[...]
            ri = ri_ref[...]
            sl = jnp.max(jnp.where(rd == tau[:, None], ri, -_MAXI), axis=1)
            ev = (rd == tau[:, None]) & (ri == sl[:, None]) & improve[:, None]
            rd_ref[...] = jnp.where(ev, mc[:, None], rd)
            ri_ref[...] = jnp.where(ev, gi[:, None], ri)
            rv_ref[...] = jnp.where(ev, vsel[:, None], rv_ref[...])
            dm_ref[...] = jnp.where(hit, jnp.inf, dm)

        return anyimp.astype(jnp.int32)

    jax.lax.while_loop(cond, body, jnp.int32(1))

    @pl.when(c == n_chunks - 1)
    def _finish():
        rd = rd_ref[...]
        w = rd / (rd + _EPS)
        out_ref[...] = (jnp.sum(w * rv_ref[...], axis=1) / float(_K))[:, None]


def _impl(queries, keys, values, chunk, qb, interpret=False):
    b, dim = queries.shape
    n = keys.shape[0]
    n_chunks = -(-n // chunk)
    n_pad = n_chunks * chunk
    kp = jnp.pad(keys, ((0, n_pad - n), (0, 0)))
    vp = jnp.pad(values[:, 0], (0, n_pad - n)).reshape(n_chunks, 1, chunk)
    n_qb = b // qb

    kfn = functools.partial(_knn_kernel, n_valid=n, chunk=chunk,
                            n_chunks=n_chunks)
    out = pl.pallas_call(
        kfn,
        grid=(n_qb, n_chunks),
        in_specs=[
            pl.BlockSpec((qb, dim), lambda q, c: (q, 0)),
            pl.BlockSpec((chunk, dim), lambda q, c: (c, 0)),
            pl.BlockSpec((1, 1, chunk), lambda q, c: (c, 0, 0)),
        ],
        out_specs=pl.BlockSpec((qb, 1), lambda q, c: (q, 0)),
        out_shape=jax.ShapeDtypeStruct((b, 1), jnp.float32),
        scratch_shapes=[
            pltpu.VMEM((qb, chunk), jnp.float32),
            pltpu.VMEM((qb, _K), jnp.float32),
            pltpu.VMEM((qb, _K), jnp.int32),
            pltpu.VMEM((qb, _K), jnp.float32),
        ],
        compiler_params=pltpu.CompilerParams(
            dimension_semantics=("parallel", "arbitrary")),
        interpret=interpret,
    )(queries, kp, vp)
    return out


def kernel(queries, keys, values):
    return _impl(queries, keys, values, chunk=2048, qb=1024)
